# baseline (speedup 1.0000x reference)
import functools
from math import sqrt

import jax
import jax.numpy as jnp
from jax.experimental import pallas as pl
from jax.experimental.pallas import tpu as pltpu

B, L, DIM, H, D = 2, 2048, 2, 12, 64
U = 24
C = 48
NCHUNK = 16
NPAIR = DIM * H // 2
NEG = -3.0e38
BIG = 4 * L
HIGHEST = jax.lax.Precision.HIGHEST


def _kernel_body(q_ref, k_ref, v_ref, lo_ref, hi_ref, out_ref,
                 s_ref, sc_ref, m_ref, cand_ref, sel_ref, pos_ref):
    f32 = jnp.float32
    col = jax.lax.broadcasted_iota(jnp.int32, (128, 128), 1)
    col_valid = col < U
    colC = jax.lax.broadcasted_iota(jnp.int32, (C, 128), 1)
    colC_valid = colC < U

    def chunk_body(c, _):
        lo_c = lo_ref[pl.ds(c * 128, 128), :]
        hi_c = hi_ref[pl.ds(c * 128, 128), :]
        for sub in range(2):
            c0, c1 = sub * D, (sub + 1) * D
            qc = q_ref[pl.ds(c * 128, 128), c0:c1]
            s_ref[sub, :, :] = jax.lax.dot_general(
                qc, k_ref[:, c0:c1], (((1,), (1,)), ((), ())),
                preferred_element_type=f32)
            g = jnp.zeros((128, 128), f32)
            for t in range(NCHUNK):
                gt = jnp.take_along_axis(
                    s_ref[sub, :, t * 128:(t + 1) * 128], lo_c, axis=1)
                g = jnp.where(hi_c == t, gt, g)
            gmax = jnp.max(jnp.where(col_valid, g, NEG), axis=1)
            gsum = jnp.sum(g, axis=1)
            m_ref[sub, c, :] = gmax - gsum / float(L)
        return 0

    jax.lax.fori_loop(0, NCHUNK, chunk_body, 0, unroll=False)

    flat = (jax.lax.broadcasted_iota(jnp.int32, (NCHUNK, 128), 0) * 128
            + jax.lax.broadcasted_iota(jnp.int32, (NCHUNK, 128), 1))
    iotaC = jax.lax.broadcasted_iota(jnp.int32, (C, 1), 0)

    def topc_body(t, carry):
        m0, m1, cq0, cq1 = carry
        outs = []
        for sub, (m_val, cq) in enumerate(((m0, cq0), (m1, cq1))):
            mx = jnp.max(m_val)
            i = jnp.min(jnp.where(m_val == mx, flat, BIG))
            cand_ref[sub, t] = i
            outs.append(jnp.where(flat == i, NEG, m_val))
            outs.append(jnp.where(iotaC == t, i, cq))
        return tuple(outs[::2]) + tuple(outs[1::2])

    _, _, candq0, candq1 = jax.lax.fori_loop(
        0, C, topc_body,
        (m_ref[0, :, :], m_ref[1, :, :],
         jnp.zeros((C, 1), jnp.int32), jnp.zeros((C, 1), jnp.int32)),
        unroll=False)

    me_list = []
    for sub, candq in ((0, candq0), (1, candq1)):
        c0, c1 = sub * D, (sub + 1) * D
        qrows, lorows, hirows = [], [], []
        for j in range(C):
            cj = cand_ref[sub, j]
            qrows.append(q_ref[pl.ds(cj, 1), c0:c1])
            lorows.append(lo_ref[pl.ds(cj, 1), :])
            hirows.append(hi_ref[pl.ds(cj, 1), :])
        q_cand = jnp.concatenate(qrows, axis=0)
        lo_cand = jnp.concatenate(lorows, axis=0)
        hi_cand = jnp.concatenate(hirows, axis=0)
        sc_ref[sub, :, :] = jax.lax.dot_general(
            q_cand, k_ref[:, c0:c1], (((1,), (1,)), ((), ())),
            precision=HIGHEST, preferred_element_type=f32)
        g = jnp.zeros((C, 128), f32)
        for t in range(NCHUNK):
            gt = jnp.take_along_axis(
                sc_ref[sub, :, t * 128:(t + 1) * 128], lo_cand, axis=1)
            g = jnp.where(hi_cand == t, gt, g)
        gmax = jnp.max(jnp.where(colC_valid, g, NEG), axis=1, keepdims=True)
        gsum = jnp.sum(g, axis=1, keepdims=True)
        me_list.append(gmax - gsum / float(L))

    def topu_body(t, carry):
        me0, me1 = carry
        outs = []
        for sub, (me, candq) in enumerate(((me0, candq0), (me1, candq1))):
            mx = jnp.max(me)
            eq = me == mx
            qi = jnp.min(jnp.where(eq, candq, BIG))
            pos = jnp.min(jnp.where(candq == qi, iotaC, BIG))
            sel_ref[sub, t] = qi
            pos_ref[sub, t] = pos
            outs.append(jnp.where(iotaC == pos, NEG, me))
        return tuple(outs)

    jax.lax.fori_loop(0, U, topu_body, (me_list[0], me_list[1]), unroll=False)

    for sub in range(2):
        c0, c1 = sub * D, (sub + 1) * D
        V_val = v_ref[:, c0:c1]
        srows = []
        for t in range(U):
            srows.append(sc_ref[sub, pl.ds(pos_ref[sub, t], 1), :])
        scores = jnp.concatenate(srows, axis=0) * (1.0 / sqrt(D))
        smax = jnp.max(scores, axis=1, keepdims=True)
        unnorm = jnp.exp(scores - smax)
        attn = unnorm / jnp.sum(unnorm, axis=1, keepdims=True)
        out24 = jax.lax.dot_general(
            attn, V_val, (((1,), (0,)), ((), ())),
            precision=HIGHEST, preferred_element_type=f32)
        vmean = jnp.mean(V_val, axis=0, keepdims=True)
        out_ref[sub, :, :] = jnp.broadcast_to(vmean, (L, D))
        for t in range(U):
            out_ref[sub, pl.ds(sel_ref[sub, t], 1), :] = out24[t:t + 1, :]


@jax.jit
def kernel(queries, keys, values, attn_mask):
    del attn_mask
    idx = jax.random.randint(jax.random.key(42), (L, U), 0, L)
    lo = jnp.concatenate(
        [idx % 128, jnp.zeros((L, 128 - U), jnp.int32)], axis=1)
    hi = jnp.concatenate(
        [idx // 128, jnp.full((L, 128 - U), -1, jnp.int32)], axis=1)

    qkv_spec = pl.BlockSpec(
        (L, 128), lambda sp: (sp // NPAIR, sp % NPAIR))
    idx_spec = pl.BlockSpec((L, 128), lambda sp: (0, 0))
    out_spec = pl.BlockSpec((2, L, D), lambda sp: (sp, 0, 0))

    def flat(x):
        return x.reshape(B * L, DIM * H * D)

    out = pl.pallas_call(
        _kernel_body,
        grid=(B * NPAIR,),
        in_specs=[qkv_spec, qkv_spec, qkv_spec, idx_spec, idx_spec],
        out_specs=out_spec,
        out_shape=jax.ShapeDtypeStruct((B * DIM * H, L, D), jnp.float32),
        scratch_shapes=[
            pltpu.VMEM((2, 128, L), jnp.float32),
            pltpu.VMEM((2, C, L), jnp.float32),
            pltpu.VMEM((2, NCHUNK, 128), jnp.float32),
            pltpu.SMEM((2, C), jnp.int32),
            pltpu.SMEM((2, U), jnp.int32),
            pltpu.SMEM((2, U), jnp.int32),
        ],
    )(flat(queries), flat(keys), flat(values), lo, hi)
    return out.reshape(B, DIM, H, L, D)


# speedup vs baseline: 1.2909x; 1.2909x over previous
import functools
from math import sqrt

import jax
import jax.numpy as jnp
from jax.experimental import pallas as pl
from jax.experimental.pallas import tpu as pltpu

B, L, DIM, H, D = 2, 2048, 2, 12, 64
U = 24
NCHUNK = 16
NPAIR = DIM * H // 2
NEG = -3.0e38
BIG = 4 * L
HIGHEST = jax.lax.Precision.HIGHEST


def _kernel_body(q_ref, k_ref, v_ref, lo_ref, hi_ref, out_ref,
                 s_ref, m_ref, sel_ref):
    f32 = jnp.float32
    col = jax.lax.broadcasted_iota(jnp.int32, (128, 128), 1)
    col_valid = col < U

    def chunk_body(c, _):
        lo_c = lo_ref[pl.ds(c * 128, 128), :]
        hi_c = hi_ref[pl.ds(c * 128, 128), :]
        for sub in range(2):
            c0, c1 = sub * D, (sub + 1) * D
            qc = q_ref[pl.ds(c * 128, 128), c0:c1]
            s_ref[sub, :, :] = jax.lax.dot_general(
                qc, k_ref[:, c0:c1], (((1,), (1,)), ((), ())),
                precision=HIGHEST, preferred_element_type=f32)
            terms = []
            for t in range(NCHUNK):
                gt = jnp.take_along_axis(
                    s_ref[sub, :, t * 128:(t + 1) * 128], lo_c, axis=1)
                terms.append(jnp.where(hi_c == t, gt, 0.0))
            while len(terms) > 1:
                terms = [a + b for a, b in zip(terms[::2], terms[1::2])]
            g = terms[0]
            gmax = jnp.max(jnp.where(col_valid, g, NEG), axis=1)
            gsum = jnp.sum(g, axis=1)
            m_ref[sub, c, :] = gmax - gsum / float(L)
        return 0

    jax.lax.fori_loop(0, NCHUNK, chunk_body, 0, unroll=False)

    flat = (jax.lax.broadcasted_iota(jnp.int32, (NCHUNK, 128), 0) * 128
            + jax.lax.broadcasted_iota(jnp.int32, (NCHUNK, 128), 1))

    def topk_body(t, carry):
        m0, m1 = carry
        outs = []
        for sub, m_val in enumerate((m0, m1)):
            mx = jnp.max(m_val)
            i = jnp.min(jnp.where(m_val == mx, flat, BIG))
            sel_ref[sub, t] = i
            outs.append(jnp.where(flat == i, NEG, m_val))
        return tuple(outs)

    jax.lax.fori_loop(0, U, topk_body, (m_ref[0, :, :], m_ref[1, :, :]),
                      unroll=False)

    for sub in range(2):
        c0, c1 = sub * D, (sub + 1) * D
        V_val = v_ref[:, c0:c1]
        rows = []
        for t in range(U):
            rows.append(q_ref[pl.ds(sel_ref[sub, t], 1), c0:c1])
        q_sel = jnp.concatenate(rows, axis=0)
        scores = jax.lax.dot_general(
            q_sel, k_ref[:, c0:c1], (((1,), (1,)), ((), ())),
            precision=HIGHEST, preferred_element_type=f32) * (1.0 / sqrt(D))
        smax = jnp.max(scores, axis=1, keepdims=True)
        unnorm = jnp.exp(scores - smax)
        attn = unnorm / jnp.sum(unnorm, axis=1, keepdims=True)
        out24 = jax.lax.dot_general(
            attn, V_val, (((1,), (0,)), ((), ())),
            preferred_element_type=f32)
        vmean = jnp.mean(V_val, axis=0, keepdims=True)
        out_ref[sub, :, :] = jnp.broadcast_to(vmean, (L, D))
        for t in range(U):
            out_ref[sub, pl.ds(sel_ref[sub, t], 1), :] = out24[t:t + 1, :]


@jax.jit
def kernel(queries, keys, values, attn_mask):
    del attn_mask
    idx = jax.random.randint(jax.random.key(42), (L, U), 0, L)
    lo = jnp.concatenate(
        [idx % 128, jnp.zeros((L, 128 - U), jnp.int32)], axis=1)
    hi = jnp.concatenate(
        [idx // 128, jnp.full((L, 128 - U), -1, jnp.int32)], axis=1)

    qkv_spec = pl.BlockSpec(
        (L, 128), lambda sp: (sp // NPAIR, sp % NPAIR))
    idx_spec = pl.BlockSpec((L, 128), lambda sp: (0, 0))
    out_spec = pl.BlockSpec((2, L, D), lambda sp: (sp, 0, 0))

    def flat(x):
        return x.reshape(B * L, DIM * H * D)

    out = pl.pallas_call(
        _kernel_body,
        grid=(B * NPAIR,),
        in_specs=[qkv_spec, qkv_spec, qkv_spec, idx_spec, idx_spec],
        out_specs=out_spec,
        out_shape=jax.ShapeDtypeStruct((B * DIM * H, L, D), jnp.float32),
        scratch_shapes=[
            pltpu.VMEM((2, 128, L), jnp.float32),
            pltpu.VMEM((2, NCHUNK, 128), jnp.float32),
            pltpu.SMEM((2, U), jnp.int32),
        ],
    )(flat(queries), flat(keys), flat(values), lo, hi)
    return out.reshape(B, DIM, H, L, D)
